# baseline (speedup 1.0000x reference)
import functools
import math

import jax
import jax.numpy as jnp
from jax import lax
from jax.experimental import pallas as pl
from jax.experimental.pallas import tpu as pltpu
from jax.experimental.pallas import tpu_sc as plsc

N = 10000
E = 320000
D = 128
ED = 16
C = 128
QC = C + ED
QW = 80
KW = C // 2
KVW = KW + C

NC = 2
NS = 16
NW = NC * NS

CH = 32
NCHUNK = E // CH
ROW = 160
ZCH = 16
NZC = N // ZCH

TB = 1000
GRID = N // TB

_RSQRT_C = 1.0 / math.sqrt(float(C))


def _proj_body(h, qW, qb, kW, kb, vW, vb, sW, sb, eW):
    qs = (jnp.dot(h, qW.T, preferred_element_type=jnp.float32) + qb) * _RSQRT_C
    k = jnp.dot(h, kW.T, preferred_element_type=jnp.float32) + kb
    v = jnp.dot(h, vW.T, preferred_element_type=jnp.float32) + vb
    skip = jnp.dot(h, sW.T, preferred_element_type=jnp.float32) + sb
    qe = jnp.dot(qs, eW, preferred_element_type=jnp.float32)
    return jnp.concatenate([qs, qe], axis=1), k, v, skip


def _tc_pre_body(x_ref, W1_ref, b1_ref, qW_ref, qb_ref, kW_ref, kb_ref,
                 vW_ref, vb_ref, sW_ref, sb_ref, eW_ref,
                 qc_ref, k_ref, v_ref, skip_ref):
    x = x_ref[...]
    h = jnp.maximum(
        jnp.dot(x, W1_ref[...].T, preferred_element_type=jnp.float32)
        + b1_ref[...], 0.0)
    qc, k, v, skip = _proj_body(
        h, qW_ref[...], qb_ref[...], kW_ref[...], kb_ref[...], vW_ref[...],
        vb_ref[...], sW_ref[...], sb_ref[...], eW_ref[...])
    qc_ref[...] = qc
    k_ref[...] = k
    v_ref[...] = v
    skip_ref[...] = skip


def _norm_block(u, eW, skip):
    usum = u[0] + u[1]
    dinv = 1.0 / (usum[:, 144:145] + 1e-16)
    msg = usum[:, 0:128] * dinv
    z = usum[:, 128:144] * dinv
    h1 = msg + jnp.dot(z, eW.T, preferred_element_type=jnp.float32) + skip
    return jnp.maximum(h1, 0.0)


def _tc_mid_body(u_ref, e1W_ref, skip1_ref, W2_ref, b2_ref,
                 qW_ref, qb_ref, kW_ref, kb_ref, vW_ref, vb_ref,
                 sW_ref, sb_ref, e2W_ref,
                 qc_ref, k_ref, v_ref, skip_ref):
    h1 = _norm_block(u_ref[...], e1W_ref[...], skip1_ref[...])
    h = jnp.maximum(
        jnp.dot(h1, W2_ref[...].T, preferred_element_type=jnp.float32)
        + b2_ref[...], 0.0)
    qc, k, v, skip = _proj_body(
        h, qW_ref[...], qb_ref[...], kW_ref[...], kb_ref[...], vW_ref[...],
        vb_ref[...], sW_ref[...], sb_ref[...], e2W_ref[...])
    qc_ref[...] = qc
    k_ref[...] = k
    v_ref[...] = v
    skip_ref[...] = skip


def _tc_post_body(u_ref, e2W_ref, skip2_ref, W3_ref, b3_ref, out_ref):
    h = _norm_block(u_ref[...], e2W_ref[...], skip2_ref[...])
    out_ref[...] = (jnp.sum(h * W3_ref[...], axis=1, keepdims=True)
                    + b3_ref[0, 0])


def _full(shape):
    return pl.BlockSpec(shape, lambda i: tuple(0 for _ in shape))


_W_SPECS = [
    _full((C, C)), _full((1, C)),
    _full((C, C)), _full((1, C)),
    _full((C, C)), _full((1, C)),
    _full((C, C)), _full((1, C)),
    _full((C, ED)),
]

_PROJ_OUT_SPECS = [
    pl.BlockSpec((TB, QC), lambda i: (i, 0)),
    pl.BlockSpec((TB, C), lambda i: (i, 0)),
    pl.BlockSpec((TB, C), lambda i: (i, 0)),
    pl.BlockSpec((TB, C), lambda i: (i, 0)),
]

_PROJ_OUT_SHAPES = [
    jax.ShapeDtypeStruct((N, QC), jnp.float32),
    jax.ShapeDtypeStruct((N, C), jnp.float32),
    jax.ShapeDtypeStruct((N, C), jnp.float32),
    jax.ShapeDtypeStruct((N, C), jnp.float32),
]

_tc_pre = pl.pallas_call(
    _tc_pre_body,
    grid=(GRID,),
    in_specs=[pl.BlockSpec((TB, D), lambda i: (i, 0)),
              _full((C, D)), _full((1, C))] + _W_SPECS,
    out_specs=_PROJ_OUT_SPECS,
    out_shape=_PROJ_OUT_SHAPES,
)

_tc_mid = pl.pallas_call(
    _tc_mid_body,
    grid=(GRID,),
    in_specs=[pl.BlockSpec((NC, TB, ROW), lambda i: (0, i, 0)),
              _full((C, ED)),
              pl.BlockSpec((TB, C), lambda i: (i, 0)),
              _full((C, C)), _full((1, C))] + _W_SPECS,
    out_specs=_PROJ_OUT_SPECS,
    out_shape=_PROJ_OUT_SHAPES,
)

_tc_post = pl.pallas_call(
    _tc_post_body,
    grid=(GRID,),
    in_specs=[pl.BlockSpec((NC, TB, ROW), lambda i: (0, i, 0)),
              _full((C, ED)),
              pl.BlockSpec((TB, C), lambda i: (i, 0)),
              _full((1, C)), _full((1, 1))],
    out_specs=pl.BlockSpec((TB, 1), lambda i: (i, 0)),
    out_shape=jax.ShapeDtypeStruct((N, 1), jnp.float32),
)


_BASE_CHUNKS = NCHUNK // NW
_EXTRA = NCHUNK - _BASE_CHUNKS * NW


def _sc_edge_body(qc_hbm, kv_hbm, ei3_hbm, ea_hbm,
                  out_hbm,
                  idxs, qrows2, kvrows2, eav2, exv, urow2,
                  uacc, semg0, semg1, sems):
    cid = lax.axis_index("c")
    sid = lax.axis_index("s")
    wid = sid * NC + cid

    iot = lax.iota(jnp.int32, 16)
    zeros16 = jnp.zeros((16,), jnp.float32)
    onehot0 = jnp.where(iot == 0, 1.0, 0.0).astype(jnp.float32)
    semg = (semg0, semg1)

    def _zero_row(i, carry):
        for t in range(ROW // 16):
            urow2[0, i, pl.ds(16 * t, 16)] = zeros16
        return carry

    lax.fori_loop(0, ZCH, _zero_row, 0)

    def _zero_chunk(i, carry):
        off = pl.multiple_of((sid + NS * i) * ZCH, 8)
        pltpu.sync_copy(urow2.at[0, pl.ds(0, ZCH)],
                        uacc.at[pl.ds(off, ZCH)])
        return carry

    nzc_mine = (NZC - 1 - sid) // NS + 1
    lax.fori_loop(0, nzc_mine, _zero_chunk, 0)
    plsc.subcore_barrier()

    start = wid * _BASE_CHUNKS + jnp.minimum(wid, _EXTRA)
    nch = _BASE_CHUNKS + jnp.where(wid < _EXTRA, 1, 0)

    def _sidx(ci, which):
        return idxs.at[(ci >> 3) & 1, ci & 7, which]

    maskhi = jnp.full((16,), -65536, jnp.int32)

    def _unpack(w):
        lo = plsc.bitcast(jnp.left_shift(w, 16), jnp.float32)
        hi = plsc.bitcast(jnp.bitwise_and(w, maskhi), jnp.float32)
        return lo, hi

    def _issue(ci, b):
        pltpu.async_copy(qc_hbm.at[_sidx(ci, 1)], qrows2.at[b], semg[b])
        pltpu.async_copy(kv_hbm.at[_sidx(ci, 0)], kvrows2.at[b], semg[b])
        pltpu.async_copy(ea_hbm.at[pl.ds((start + ci) * CH, CH)],
                         eav2.at[b], semg[b])

    def _process(ci, b):
        @pl.when(ci >= 2)
        def _():
            pltpu.make_async_copy(out_hbm.at[cid, pl.ds(0, CH)],
                                  urow2.at[b], sems).wait()

        pltpu.make_async_copy(qc_hbm.at[pl.ds(0, CH)], qrows2.at[b],
                              semg[b]).wait()
        pltpu.make_async_copy(kv_hbm.at[pl.ds(0, CH)], kvrows2.at[b],
                              semg[b]).wait()
        pltpu.make_async_copy(ea_hbm.at[pl.ds(0, CH)], eav2.at[b],
                              semg[b]).wait()

        def _group(g, gcarry):
            e16 = g * 16 + iot
            acc = zeros16
            for t in range(KW):
                idx = jnp.full((16,), t, jnp.int32)
                qlo, qhi = _unpack(plsc.load_gather(qrows2.at[b],
                                                    [e16, idx]))
                klo, khi = _unpack(plsc.load_gather(kvrows2.at[b],
                                                    [e16, idx]))
                acc = acc + qlo * klo + qhi * khi
            for t in range(ED // 2):
                idx = jnp.full((16,), KW + t, jnp.int32)
                qlo, qhi = _unpack(plsc.load_gather(qrows2.at[b],
                                                    [e16, idx]))
                elo = plsc.load_gather(
                    eav2.at[b], [e16, jnp.full((16,), 2 * t, jnp.int32)])
                ehi = plsc.load_gather(
                    eav2.at[b], [e16, jnp.full((16,), 2 * t + 1, jnp.int32)])
                acc = acc + qlo * elo + qhi * ehi
            exv[pl.ds(g * 16, 16)] = jnp.exp(acc)
            return gcarry

        lax.fori_loop(0, CH // 16, _group, 0)

        for j in range(CH):
            exs = plsc.load_gather(exv, [jnp.full((16,), j, jnp.int32)])
            for t in range(C // 16):
                vt = plsc.bitcast(kvrows2[b, j, pl.ds(KW + 16 * t, 16)],
                                  jnp.float32)
                urow2[b, j, pl.ds(16 * t, 16)] = exs * vt
            urow2[b, j, pl.ds(C, 16)] = exs * eav2[b, j, :]
            urow2[b, j, pl.ds(C + ED, 16)] = exs * onehot0

        pltpu.async_copy(urow2.at[b], uacc.at[_sidx(ci, 1)], sems,
                         add=True)

        @pl.when(jnp.logical_and((ci & 7) == 6, ci + 2 < nch))
        def _():
            nci = ci + 2
            pltpu.sync_copy(ei3_hbm.at[pl.ds(start + nci, 8)],
                            idxs.at[(nci >> 3) & 1])

        @pl.when(ci + 2 < nch)
        def _():
            _issue(ci + 2, b)

    pltpu.sync_copy(ei3_hbm.at[pl.ds(start, 8)], idxs.at[0])
    _issue(0, 0)
    _issue(1, 1)

    def _chunk(ci, carry):
        even = lax.rem(ci, 2) == 0

        @pl.when(even)
        def _():
            _process(ci, 0)

        @pl.when(jnp.logical_not(even))
        def _():
            _process(ci, 1)

        return carry

    lax.fori_loop(0, nch, _chunk, 0)

    pltpu.make_async_copy(out_hbm.at[cid, pl.ds(0, CH)], urow2.at[0],
                          sems).wait()
    pltpu.make_async_copy(out_hbm.at[cid, pl.ds(0, CH)], urow2.at[1],
                          sems).wait()

    plsc.subcore_barrier()

    def _out_chunk(i, carry):
        off = pl.multiple_of((sid + NS * i) * ZCH, 8)
        pltpu.sync_copy(uacc.at[pl.ds(off, ZCH)],
                        out_hbm.at[cid, pl.ds(off, ZCH)])
        return carry

    lax.fori_loop(0, nzc_mine, _out_chunk, 0)


_sc_edge = pl.kernel(
    _sc_edge_body,
    out_type=jax.ShapeDtypeStruct((NC, N, ROW), jnp.float32),
    mesh=plsc.VectorSubcoreMesh(core_axis_name="c", subcore_axis_name="s",
                                num_cores=NC, num_subcores=NS),
    compiler_params=pltpu.CompilerParams(needs_layout_passes=False,
                                         use_tc_tiling_on_sc=False),
    scratch_types=[
        pltpu.VMEM((2, 8, 2, CH), jnp.int32),
        pltpu.VMEM((2, CH, QW), jnp.int32),
        pltpu.VMEM((2, CH, KVW), jnp.int32),
        pltpu.VMEM((2, CH, ED), jnp.float32),
        pltpu.VMEM((CH,), jnp.float32),
        pltpu.VMEM((2, CH, ROW), jnp.float32),
        pltpu.VMEM_SHARED((N, ROW), jnp.float32),
        pltpu.SemaphoreType.DMA,
        pltpu.SemaphoreType.DMA,
        pltpu.SemaphoreType.DMA,
    ],
)


def _pack_bf16(qc, k, v):
    qcp = jnp.pad(qc.astype(jnp.bfloat16), ((0, 0), (0, 2 * QW - QC)))
    qcb = lax.bitcast_convert_type(qcp.reshape(N, QW, 2), jnp.int32)
    kb = lax.bitcast_convert_type(
        k.astype(jnp.bfloat16).reshape(N, KW, 2), jnp.int32)
    kvb = jnp.concatenate([kb, lax.bitcast_convert_type(v, jnp.int32)],
                          axis=1)
    return qcb, kvb


def kernel(x, edge_index, edge_attr,
           W1, b1, q1W, q1b, k1W, k1b, v1W, v1b, e1W, s1W, s1b,
           W2, b2, q2W, q2b, k2W, k2b, v2W, v2b, e2W, s2W, s2b, W3, b3):
    ei3 = jnp.transpose(edge_index.reshape(2, NCHUNK, CH), (1, 0, 2))

    qc1, k1, v1, skip1 = _tc_pre(
        x, W1, b1.reshape(1, C), q1W, q1b.reshape(1, C), k1W,
        k1b.reshape(1, C), v1W, v1b.reshape(1, C), s1W, s1b.reshape(1, C),
        e1W)

    u1 = _sc_edge(*_pack_bf16(qc1, k1, v1), ei3, edge_attr)

    qc2, k2, v2, skip2 = _tc_mid(
        u1, e1W, skip1, W2, b2.reshape(1, C), q2W, q2b.reshape(1, C), k2W,
        k2b.reshape(1, C), v2W, v2b.reshape(1, C), s2W, s2b.reshape(1, C),
        e2W)

    u2 = _sc_edge(*_pack_bf16(qc2, k2, v2), ei3, edge_attr)

    out = _tc_post(u2, e2W, skip2, W3, b3.reshape(1, 1))
    return out.reshape(N)


# speedup vs baseline: 1.0361x; 1.0361x over previous
import functools
import math

import jax
import jax.numpy as jnp
from jax import lax
from jax.experimental import pallas as pl
from jax.experimental.pallas import tpu as pltpu
from jax.experimental.pallas import tpu_sc as plsc

N = 10000
E = 320000
D = 128
ED = 16
C = 128
QC = C + ED
QW = 80
KW = C // 2
VW = C // 2

NC = 2
NS = 16
NW = NC * NS

CH = 32
NCHUNK = E // CH
ROW = 160
ZCH = 16
NZC = N // ZCH

TB = 1000
GRID = N // TB

_RSQRT_C = 1.0 / math.sqrt(float(C))


def _proj_body(h, qW, qb, kW, kb, vW, vb, sW, sb, eW):
    qs = (jnp.dot(h, qW.T, preferred_element_type=jnp.float32) + qb) * _RSQRT_C
    k = jnp.dot(h, kW.T, preferred_element_type=jnp.float32) + kb
    v = jnp.dot(h, vW.T, preferred_element_type=jnp.float32) + vb
    skip = jnp.dot(h, sW.T, preferred_element_type=jnp.float32) + sb
    qe = jnp.dot(qs, eW, preferred_element_type=jnp.float32)
    return jnp.concatenate([qs, qe], axis=1), k, v, skip


def _tc_pre_body(x_ref, W1_ref, b1_ref, qW_ref, qb_ref, kW_ref, kb_ref,
                 vW_ref, vb_ref, sW_ref, sb_ref, eW_ref,
                 qc_ref, k_ref, v_ref, skip_ref):
    x = x_ref[...]
    h = jnp.maximum(
        jnp.dot(x, W1_ref[...].T, preferred_element_type=jnp.float32)
        + b1_ref[...], 0.0)
    qc, k, v, skip = _proj_body(
        h, qW_ref[...], qb_ref[...], kW_ref[...], kb_ref[...], vW_ref[...],
        vb_ref[...], sW_ref[...], sb_ref[...], eW_ref[...])
    qc_ref[...] = qc
    k_ref[...] = k
    v_ref[...] = v
    skip_ref[...] = skip


def _norm_block(u, eW, skip):
    usum = u[0] + u[1]
    dinv = 1.0 / (usum[:, 144:145] + 1e-16)
    msg = usum[:, 0:128] * dinv
    z = usum[:, 128:144] * dinv
    h1 = msg + jnp.dot(z, eW.T, preferred_element_type=jnp.float32) + skip
    return jnp.maximum(h1, 0.0)


def _tc_mid_body(u_ref, e1W_ref, skip1_ref, W2_ref, b2_ref,
                 qW_ref, qb_ref, kW_ref, kb_ref, vW_ref, vb_ref,
                 sW_ref, sb_ref, e2W_ref,
                 qc_ref, k_ref, v_ref, skip_ref):
    h1 = _norm_block(u_ref[...], e1W_ref[...], skip1_ref[...])
    h = jnp.maximum(
        jnp.dot(h1, W2_ref[...].T, preferred_element_type=jnp.float32)
        + b2_ref[...], 0.0)
    qc, k, v, skip = _proj_body(
        h, qW_ref[...], qb_ref[...], kW_ref[...], kb_ref[...], vW_ref[...],
        vb_ref[...], sW_ref[...], sb_ref[...], e2W_ref[...])
    qc_ref[...] = qc
    k_ref[...] = k
    v_ref[...] = v
    skip_ref[...] = skip


def _tc_post_body(u_ref, e2W_ref, skip2_ref, W3_ref, b3_ref, out_ref):
    h = _norm_block(u_ref[...], e2W_ref[...], skip2_ref[...])
    out_ref[...] = (jnp.sum(h * W3_ref[...], axis=1, keepdims=True)
                    + b3_ref[0, 0])


def _full(shape):
    return pl.BlockSpec(shape, lambda i: tuple(0 for _ in shape))


_W_SPECS = [
    _full((C, C)), _full((1, C)),
    _full((C, C)), _full((1, C)),
    _full((C, C)), _full((1, C)),
    _full((C, C)), _full((1, C)),
    _full((C, ED)),
]

_PROJ_OUT_SPECS = [
    pl.BlockSpec((TB, QC), lambda i: (i, 0)),
    pl.BlockSpec((TB, C), lambda i: (i, 0)),
    pl.BlockSpec((TB, C), lambda i: (i, 0)),
    pl.BlockSpec((TB, C), lambda i: (i, 0)),
]

_PROJ_OUT_SHAPES = [
    jax.ShapeDtypeStruct((N, QC), jnp.float32),
    jax.ShapeDtypeStruct((N, C), jnp.float32),
    jax.ShapeDtypeStruct((N, C), jnp.float32),
    jax.ShapeDtypeStruct((N, C), jnp.float32),
]

_tc_pre = pl.pallas_call(
    _tc_pre_body,
    grid=(GRID,),
    in_specs=[pl.BlockSpec((TB, D), lambda i: (i, 0)),
              _full((C, D)), _full((1, C))] + _W_SPECS,
    out_specs=_PROJ_OUT_SPECS,
    out_shape=_PROJ_OUT_SHAPES,
)

_tc_mid = pl.pallas_call(
    _tc_mid_body,
    grid=(GRID,),
    in_specs=[pl.BlockSpec((NC, TB, ROW), lambda i: (0, i, 0)),
              _full((C, ED)),
              pl.BlockSpec((TB, C), lambda i: (i, 0)),
              _full((C, C)), _full((1, C))] + _W_SPECS,
    out_specs=_PROJ_OUT_SPECS,
    out_shape=_PROJ_OUT_SHAPES,
)

_tc_post = pl.pallas_call(
    _tc_post_body,
    grid=(GRID,),
    in_specs=[pl.BlockSpec((NC, TB, ROW), lambda i: (0, i, 0)),
              _full((C, ED)),
              pl.BlockSpec((TB, C), lambda i: (i, 0)),
              _full((1, C)), _full((1, 1))],
    out_specs=pl.BlockSpec((TB, 1), lambda i: (i, 0)),
    out_shape=jax.ShapeDtypeStruct((N, 1), jnp.float32),
)


_BASE_CHUNKS = NCHUNK // NW
_EXTRA = NCHUNK - _BASE_CHUNKS * NW


def _sc_edge_body(qc_hbm, k_hbm, v_hbm, ei3_hbm, ea_hbm,
                  out_hbm,
                  idxs, qrows2, krows2, vrows, eav2, exv, urow2,
                  uacc, semg0, semg1, semv, sems):
    cid = lax.axis_index("c")
    sid = lax.axis_index("s")
    wid = sid * NC + cid

    iot = lax.iota(jnp.int32, 16)
    zeros16 = jnp.zeros((16,), jnp.float32)
    onehot0 = jnp.where(iot == 0, 1.0, 0.0).astype(jnp.float32)
    semg = (semg0, semg1)

    def _zero_row(i, carry):
        for t in range(ROW // 16):
            urow2[0, i, pl.ds(16 * t, 16)] = zeros16
        return carry

    lax.fori_loop(0, ZCH, _zero_row, 0)

    def _zero_chunk(i, carry):
        off = pl.multiple_of((sid + NS * i) * ZCH, 8)
        pltpu.sync_copy(urow2.at[0, pl.ds(0, ZCH)],
                        uacc.at[pl.ds(off, ZCH)])
        return carry

    nzc_mine = (NZC - 1 - sid) // NS + 1
    lax.fori_loop(0, nzc_mine, _zero_chunk, 0)
    plsc.subcore_barrier()

    start = wid * _BASE_CHUNKS + jnp.minimum(wid, _EXTRA)
    nch = _BASE_CHUNKS + jnp.where(wid < _EXTRA, 1, 0)

    def _sidx(ci, which):
        return idxs.at[(ci >> 3) & 1, ci & 7, which]

    maskhi = jnp.full((16,), -65536, jnp.int32)

    def _unpack(w):
        lo = plsc.bitcast(jnp.left_shift(w, 16), jnp.float32)
        hi = plsc.bitcast(jnp.bitwise_and(w, maskhi), jnp.float32)
        return lo, hi

    def _issue_kea(ci, b):
        pltpu.async_copy(k_hbm.at[_sidx(ci, 0)], krows2.at[b], semg[b])
        pltpu.async_copy(ea_hbm.at[pl.ds((start + ci) * CH, CH)],
                         eav2.at[b], semg[b])

    def _process(ci, b):
        @pl.when(ci >= 2)
        def _():
            pltpu.make_async_copy(out_hbm.at[cid, pl.ds(0, CH)],
                                  urow2.at[b], sems).wait()

        pltpu.async_copy(v_hbm.at[_sidx(ci, 0)], vrows, semv)
        pltpu.make_async_copy(qc_hbm.at[pl.ds(0, CH)], qrows2.at[b],
                              semg[b]).wait()
        pltpu.make_async_copy(k_hbm.at[pl.ds(0, CH)], krows2.at[b],
                              semg[b]).wait()
        pltpu.make_async_copy(ea_hbm.at[pl.ds(0, CH)], eav2.at[b],
                              semg[b]).wait()

        def _group(g, gcarry):
            e16 = g * 16 + iot
            acc = zeros16
            for t in range(KW):
                idx = jnp.full((16,), t, jnp.int32)
                qlo, qhi = _unpack(plsc.load_gather(qrows2.at[b],
                                                    [e16, idx]))
                klo, khi = _unpack(plsc.load_gather(krows2.at[b],
                                                    [e16, idx]))
                acc = acc + qlo * klo + qhi * khi
            for t in range(ED // 2):
                idx = jnp.full((16,), KW + t, jnp.int32)
                qlo, qhi = _unpack(plsc.load_gather(qrows2.at[b],
                                                    [e16, idx]))
                elo = plsc.load_gather(
                    eav2.at[b], [e16, jnp.full((16,), 2 * t, jnp.int32)])
                ehi = plsc.load_gather(
                    eav2.at[b], [e16, jnp.full((16,), 2 * t + 1, jnp.int32)])
                acc = acc + qlo * elo + qhi * ehi
            exv[pl.ds(g * 16, 16)] = jnp.exp(acc)
            return gcarry

        lax.fori_loop(0, CH // 16, _group, 0)

        @pl.when(jnp.logical_and((ci & 7) == 6, ci + 2 < nch))
        def _():
            nci = ci + 2
            pltpu.sync_copy(ei3_hbm.at[pl.ds(start + nci, 8)],
                            idxs.at[(nci >> 3) & 1])

        @pl.when(ci + 2 < nch)
        def _():
            pltpu.async_copy(qc_hbm.at[_sidx(ci + 2, 1)], qrows2.at[b],
                             semg[b])

        pltpu.make_async_copy(v_hbm.at[pl.ds(0, CH)], vrows, semv).wait()

        for j in range(CH):
            exs = plsc.load_gather(exv, [jnp.full((16,), j, jnp.int32)])
            for t in range(VW // 16):
                vlo, vhi = _unpack(vrows[j, pl.ds(16 * t, 16)])
                urow2[b, j, pl.ds(32 * t, 16)] = exs * vlo
                urow2[b, j, pl.ds(32 * t + 16, 16)] = exs * vhi
            urow2[b, j, pl.ds(C, 16)] = exs * eav2[b, j, :]
            urow2[b, j, pl.ds(C + ED, 16)] = exs * onehot0

        pltpu.async_copy(urow2.at[b], uacc.at[_sidx(ci, 1)], sems,
                         add=True)

        @pl.when(ci + 2 < nch)
        def _():
            _issue_kea(ci + 2, b)

    pltpu.sync_copy(ei3_hbm.at[pl.ds(start, 8)], idxs.at[0])
    for b0 in (0, 1):
        pltpu.async_copy(qc_hbm.at[_sidx(b0, 1)], qrows2.at[b0], semg[b0])
        _issue_kea(b0, b0)

    def _chunk(ci, carry):
        even = lax.rem(ci, 2) == 0

        @pl.when(even)
        def _():
            _process(ci, 0)

        @pl.when(jnp.logical_not(even))
        def _():
            _process(ci, 1)

        return carry

    lax.fori_loop(0, nch, _chunk, 0)

    pltpu.make_async_copy(out_hbm.at[cid, pl.ds(0, CH)], urow2.at[0],
                          sems).wait()
    pltpu.make_async_copy(out_hbm.at[cid, pl.ds(0, CH)], urow2.at[1],
                          sems).wait()

    plsc.subcore_barrier()

    def _out_chunk(i, carry):
        off = pl.multiple_of((sid + NS * i) * ZCH, 8)
        pltpu.sync_copy(uacc.at[pl.ds(off, ZCH)],
                        out_hbm.at[cid, pl.ds(off, ZCH)])
        return carry

    lax.fori_loop(0, nzc_mine, _out_chunk, 0)


_sc_edge = pl.kernel(
    _sc_edge_body,
    out_type=jax.ShapeDtypeStruct((NC, N, ROW), jnp.float32),
    mesh=plsc.VectorSubcoreMesh(core_axis_name="c", subcore_axis_name="s",
                                num_cores=NC, num_subcores=NS),
    compiler_params=pltpu.CompilerParams(needs_layout_passes=False,
                                         use_tc_tiling_on_sc=False),
    scratch_types=[
        pltpu.VMEM((2, 8, 2, CH), jnp.int32),
        pltpu.VMEM((2, CH, QW), jnp.int32),
        pltpu.VMEM((2, CH, KW), jnp.int32),
        pltpu.VMEM((CH, VW), jnp.int32),
        pltpu.VMEM((2, CH, ED), jnp.float32),
        pltpu.VMEM((CH,), jnp.float32),
        pltpu.VMEM((2, CH, ROW), jnp.float32),
        pltpu.VMEM_SHARED((N, ROW), jnp.float32),
        pltpu.SemaphoreType.DMA,
        pltpu.SemaphoreType.DMA,
        pltpu.SemaphoreType.DMA,
        pltpu.SemaphoreType.DMA,
    ],
)


def _pack_bf16(qc, k, v):
    qcp = jnp.pad(qc.astype(jnp.bfloat16), ((0, 0), (0, 2 * QW - QC)))
    qcb = lax.bitcast_convert_type(qcp.reshape(N, QW, 2), jnp.int32)
    kb = lax.bitcast_convert_type(
        k.astype(jnp.bfloat16).reshape(N, KW, 2), jnp.int32)
    vp = jnp.transpose(v.astype(jnp.bfloat16).reshape(N, 4, 2, 16),
                       (0, 1, 3, 2))
    vb = lax.bitcast_convert_type(vp, jnp.int32).reshape(N, VW)
    return qcb, kb, vb


def kernel(x, edge_index, edge_attr,
           W1, b1, q1W, q1b, k1W, k1b, v1W, v1b, e1W, s1W, s1b,
           W2, b2, q2W, q2b, k2W, k2b, v2W, v2b, e2W, s2W, s2b, W3, b3):
    ei3 = jnp.transpose(edge_index.reshape(2, NCHUNK, CH), (1, 0, 2))

    qc1, k1, v1, skip1 = _tc_pre(
        x, W1, b1.reshape(1, C), q1W, q1b.reshape(1, C), k1W,
        k1b.reshape(1, C), v1W, v1b.reshape(1, C), s1W, s1b.reshape(1, C),
        e1W)

    u1 = _sc_edge(*_pack_bf16(qc1, k1, v1), ei3, edge_attr)

    qc2, k2, v2, skip2 = _tc_mid(
        u1, e1W, skip1, W2, b2.reshape(1, C), q2W, q2b.reshape(1, C), k2W,
        k2b.reshape(1, C), v2W, v2b.reshape(1, C), s2W, s2b.reshape(1, C),
        e2W)

    u2 = _sc_edge(*_pack_bf16(qc2, k2, v2), ei3, edge_attr)

    out = _tc_post(u2, e2W, skip2, W3, b3.reshape(1, 1))
    return out.reshape(N)


# speedup vs baseline: 1.1479x; 1.1079x over previous
import functools
import math

import jax
import jax.numpy as jnp
from jax import lax
from jax.experimental import pallas as pl
from jax.experimental.pallas import tpu as pltpu
from jax.experimental.pallas import tpu_sc as plsc

N = 10000
E = 320000
D = 128
ED = 16
C = 128
QC = C + ED
QW = 80
KW = C // 2
VW = C // 2
EAX = 48

NC = 2
NS = 16
NW = NC * NS

CH = 32
NCHUNK = E // CH
ROW = 160
ZCH = 16
NZC = N // ZCH

TB = 1000
GRID = N // TB

_RSQRT_C = 1.0 / math.sqrt(float(C))


def _proj_body(h, qW, qb, kW, kb, vW, vb, sW, sb, eW):
    qs = (jnp.dot(h, qW.T, preferred_element_type=jnp.float32) + qb) * _RSQRT_C
    k = jnp.dot(h, kW.T, preferred_element_type=jnp.float32) + kb
    v = jnp.dot(h, vW.T, preferred_element_type=jnp.float32) + vb
    skip = jnp.dot(h, sW.T, preferred_element_type=jnp.float32) + sb
    qe = jnp.dot(qs, eW, preferred_element_type=jnp.float32)
    return jnp.concatenate([qs, qe], axis=1), k, v, skip


def _tc_pre_body(x_ref, W1_ref, b1_ref, qW_ref, qb_ref, kW_ref, kb_ref,
                 vW_ref, vb_ref, sW_ref, sb_ref, eW_ref,
                 qc_ref, k_ref, v_ref, skip_ref):
    x = x_ref[...]
    h = jnp.maximum(
        jnp.dot(x, W1_ref[...].T, preferred_element_type=jnp.float32)
        + b1_ref[...], 0.0)
    qc, k, v, skip = _proj_body(
        h, qW_ref[...], qb_ref[...], kW_ref[...], kb_ref[...], vW_ref[...],
        vb_ref[...], sW_ref[...], sb_ref[...], eW_ref[...])
    qc_ref[...] = qc
    k_ref[...] = k
    v_ref[...] = v
    skip_ref[...] = skip


def _norm_block(u, eW, skip):
    usum = u[0] + u[1]
    dinv = 1.0 / (usum[:, 144:145] + 1e-16)
    msg = usum[:, 0:128] * dinv
    z = usum[:, 128:144] * dinv
    h1 = msg + jnp.dot(z, eW.T, preferred_element_type=jnp.float32) + skip
    return jnp.maximum(h1, 0.0)


def _tc_mid_body(u_ref, e1W_ref, skip1_ref, W2_ref, b2_ref,
                 qW_ref, qb_ref, kW_ref, kb_ref, vW_ref, vb_ref,
                 sW_ref, sb_ref, e2W_ref,
                 qc_ref, k_ref, v_ref, skip_ref):
    h1 = _norm_block(u_ref[...], e1W_ref[...], skip1_ref[...])
    h = jnp.maximum(
        jnp.dot(h1, W2_ref[...].T, preferred_element_type=jnp.float32)
        + b2_ref[...], 0.0)
    qc, k, v, skip = _proj_body(
        h, qW_ref[...], qb_ref[...], kW_ref[...], kb_ref[...], vW_ref[...],
        vb_ref[...], sW_ref[...], sb_ref[...], e2W_ref[...])
    qc_ref[...] = qc
    k_ref[...] = k
    v_ref[...] = v
    skip_ref[...] = skip


def _tc_post_body(u_ref, e2W_ref, skip2_ref, W3_ref, b3_ref, out_ref):
    h = _norm_block(u_ref[...], e2W_ref[...], skip2_ref[...])
    out_ref[...] = (jnp.sum(h * W3_ref[...], axis=1, keepdims=True)
                    + b3_ref[0, 0])


def _full(shape):
    return pl.BlockSpec(shape, lambda i: tuple(0 for _ in shape))


_W_SPECS = [
    _full((C, C)), _full((1, C)),
    _full((C, C)), _full((1, C)),
    _full((C, C)), _full((1, C)),
    _full((C, C)), _full((1, C)),
    _full((C, ED)),
]

_PROJ_OUT_SPECS = [
    pl.BlockSpec((TB, QC), lambda i: (i, 0)),
    pl.BlockSpec((TB, C), lambda i: (i, 0)),
    pl.BlockSpec((TB, C), lambda i: (i, 0)),
    pl.BlockSpec((TB, C), lambda i: (i, 0)),
]

_PROJ_OUT_SHAPES = [
    jax.ShapeDtypeStruct((N, QC), jnp.float32),
    jax.ShapeDtypeStruct((N, C), jnp.float32),
    jax.ShapeDtypeStruct((N, C), jnp.float32),
    jax.ShapeDtypeStruct((N, C), jnp.float32),
]

_tc_pre = pl.pallas_call(
    _tc_pre_body,
    grid=(GRID,),
    in_specs=[pl.BlockSpec((TB, D), lambda i: (i, 0)),
              _full((C, D)), _full((1, C))] + _W_SPECS,
    out_specs=_PROJ_OUT_SPECS,
    out_shape=_PROJ_OUT_SHAPES,
)

_tc_mid = pl.pallas_call(
    _tc_mid_body,
    grid=(GRID,),
    in_specs=[pl.BlockSpec((NC, TB, ROW), lambda i: (0, i, 0)),
              _full((C, ED)),
              pl.BlockSpec((TB, C), lambda i: (i, 0)),
              _full((C, C)), _full((1, C))] + _W_SPECS,
    out_specs=_PROJ_OUT_SPECS,
    out_shape=_PROJ_OUT_SHAPES,
)

_tc_post = pl.pallas_call(
    _tc_post_body,
    grid=(GRID,),
    in_specs=[pl.BlockSpec((NC, TB, ROW), lambda i: (0, i, 0)),
              _full((C, ED)),
              pl.BlockSpec((TB, C), lambda i: (i, 0)),
              _full((1, C)), _full((1, 1))],
    out_specs=pl.BlockSpec((TB, 1), lambda i: (i, 0)),
    out_shape=jax.ShapeDtypeStruct((N, 1), jnp.float32),
)


_BASE_CHUNKS = NCHUNK // NW
_EXTRA = NCHUNK - _BASE_CHUNKS * NW


def _sc_edge_body(qc_hbm, k_hbm, v_hbm, ei3_hbm, eax_hbm,
                  out_hbm,
                  idxs, qrows2, krows2, vrows, eav2, exv, urow2,
                  uacc, semg0, semg1, semv, sems):
    cid = lax.axis_index("c")
    sid = lax.axis_index("s")
    wid = sid * NC + cid

    iot = lax.iota(jnp.int32, 16)
    zeros16 = jnp.zeros((16,), jnp.float32)
    onehot0 = jnp.where(iot == 0, 1.0, 0.0).astype(jnp.float32)
    semg = (semg0, semg1)

    def _zero_row(i, carry):
        for t in range(ROW // 16):
            urow2[0, i, pl.ds(16 * t, 16)] = zeros16
        return carry

    lax.fori_loop(0, ZCH, _zero_row, 0)

    def _zero_chunk(i, carry):
        off = pl.multiple_of((sid + NS * i) * ZCH, 8)
        pltpu.sync_copy(urow2.at[0, pl.ds(0, ZCH)],
                        uacc.at[pl.ds(off, ZCH)])
        return carry

    nzc_mine = (NZC - 1 - sid) // NS + 1
    lax.fori_loop(0, nzc_mine, _zero_chunk, 0)
    plsc.subcore_barrier()

    start = wid * _BASE_CHUNKS + jnp.minimum(wid, _EXTRA)
    nch = _BASE_CHUNKS + jnp.where(wid < _EXTRA, 1, 0)

    def _sidx(ci, which):
        return idxs.at[(ci >> 3) & 1, ci & 7, which]

    maskhi = jnp.full((16,), -65536, jnp.int32)

    def _unpack(w):
        lo = plsc.bitcast(jnp.left_shift(w, 16), jnp.float32)
        hi = plsc.bitcast(jnp.bitwise_and(w, maskhi), jnp.float32)
        return lo, hi

    def _issue_kea(ci, b):
        pltpu.async_copy(k_hbm.at[_sidx(ci, 0)], krows2.at[b], semg[b])
        pltpu.async_copy(eax_hbm.at[pl.ds((start + ci) * CH, CH)],
                         eav2.at[b], semg[b])

    def _process(ci, b):
        @pl.when(ci >= 2)
        def _():
            pltpu.make_async_copy(out_hbm.at[cid, pl.ds(0, CH)],
                                  urow2.at[b], sems).wait()

        pltpu.async_copy(v_hbm.at[_sidx(ci, 0)], vrows, semv)
        pltpu.make_async_copy(qc_hbm.at[pl.ds(0, CH)], qrows2.at[b],
                              semg[b]).wait()
        pltpu.make_async_copy(k_hbm.at[pl.ds(0, CH)], krows2.at[b],
                              semg[b]).wait()
        pltpu.make_async_copy(eax_hbm.at[pl.ds(0, CH)], eav2.at[b],
                              semg[b]).wait()

        def _group_a(g, gcarry):
            alpha = zeros16
            for j in range(16):
                jj = g * 16 + j
                prod = zeros16
                for t in range(KW // 16):
                    qlo, qhi = _unpack(qrows2[b, jj, pl.ds(16 * t, 16)])
                    klo, khi = _unpack(krows2[b, jj, pl.ds(16 * t, 16)])
                    prod = prod + qlo * klo + qhi * khi
                qelo, qehi = _unpack(qrows2[b, jj, pl.ds(KW, 16)])
                prod = (prod + qelo * eav2[b, jj, pl.ds(0, 16)]
                        + qehi * eav2[b, jj, pl.ds(16, 16)])
                s = jnp.full((16,), jnp.sum(prod))
                alpha = jnp.where(iot == j, s, alpha)
            exv[pl.ds(g * 16, 16)] = jnp.exp(alpha)
            return gcarry

        lax.fori_loop(0, CH // 16, _group_a, 0)

        @pl.when(jnp.logical_and((ci & 7) == 6, ci + 2 < nch))
        def _():
            nci = ci + 2
            pltpu.sync_copy(ei3_hbm.at[pl.ds(start + nci, 8)],
                            idxs.at[(nci >> 3) & 1])

        @pl.when(ci + 2 < nch)
        def _():
            pltpu.async_copy(qc_hbm.at[_sidx(ci + 2, 1)], qrows2.at[b],
                             semg[b])

        pltpu.make_async_copy(v_hbm.at[pl.ds(0, CH)], vrows, semv).wait()

        def _group_b(g, gcarry):
            exg = exv[pl.ds(g * 16, 16)]
            for j in range(16):
                jj = g * 16 + j
                exs = jnp.full((16,), exg[j])
                for t in range(VW // 16):
                    vlo, vhi = _unpack(vrows[jj, pl.ds(16 * t, 16)])
                    urow2[b, jj, pl.ds(32 * t, 16)] = exs * vlo
                    urow2[b, jj, pl.ds(32 * t + 16, 16)] = exs * vhi
                urow2[b, jj, pl.ds(C, 16)] = exs * eav2[b, jj, pl.ds(32, 16)]
                urow2[b, jj, pl.ds(C + ED, 16)] = exs * onehot0
            return gcarry

        lax.fori_loop(0, CH // 16, _group_b, 0)

        pltpu.async_copy(urow2.at[b], uacc.at[_sidx(ci, 1)], sems,
                         add=True)

        @pl.when(ci + 2 < nch)
        def _():
            _issue_kea(ci + 2, b)

    pltpu.sync_copy(ei3_hbm.at[pl.ds(start, 8)], idxs.at[0])
    for b0 in (0, 1):
        pltpu.async_copy(qc_hbm.at[_sidx(b0, 1)], qrows2.at[b0], semg[b0])
        _issue_kea(b0, b0)

    def _chunk(ci, carry):
        even = lax.rem(ci, 2) == 0

        @pl.when(even)
        def _():
            _process(ci, 0)

        @pl.when(jnp.logical_not(even))
        def _():
            _process(ci, 1)

        return carry

    lax.fori_loop(0, nch, _chunk, 0)

    pltpu.make_async_copy(out_hbm.at[cid, pl.ds(0, CH)], urow2.at[0],
                          sems).wait()
    pltpu.make_async_copy(out_hbm.at[cid, pl.ds(0, CH)], urow2.at[1],
                          sems).wait()

    plsc.subcore_barrier()

    def _out_chunk(i, carry):
        off = pl.multiple_of((sid + NS * i) * ZCH, 8)
        pltpu.sync_copy(uacc.at[pl.ds(off, ZCH)],
                        out_hbm.at[cid, pl.ds(off, ZCH)])
        return carry

    lax.fori_loop(0, nzc_mine, _out_chunk, 0)


_sc_edge = pl.kernel(
    _sc_edge_body,
    out_type=jax.ShapeDtypeStruct((NC, N, ROW), jnp.float32),
    mesh=plsc.VectorSubcoreMesh(core_axis_name="c", subcore_axis_name="s",
                                num_cores=NC, num_subcores=NS),
    compiler_params=pltpu.CompilerParams(needs_layout_passes=False,
                                         use_tc_tiling_on_sc=False),
    scratch_types=[
        pltpu.VMEM((2, 8, 2, CH), jnp.int32),
        pltpu.VMEM((2, CH, QW), jnp.int32),
        pltpu.VMEM((2, CH, KW), jnp.int32),
        pltpu.VMEM((CH, VW), jnp.int32),
        pltpu.VMEM((2, CH, EAX), jnp.float32),
        pltpu.VMEM((CH,), jnp.float32),
        pltpu.VMEM((2, CH, ROW), jnp.float32),
        pltpu.VMEM_SHARED((N, ROW), jnp.float32),
        pltpu.SemaphoreType.DMA,
        pltpu.SemaphoreType.DMA,
        pltpu.SemaphoreType.DMA,
        pltpu.SemaphoreType.DMA,
    ],
)


def _pack_bf16(qc, k, v):
    q = qc[:, :C].astype(jnp.bfloat16)
    qe = qc[:, C:].astype(jnp.bfloat16)
    qb = lax.bitcast_convert_type(q.reshape(N, KW, 2), jnp.int32)
    qep = jnp.stack([qe[:, :8], qe[:, 8:]], axis=-1)
    qeb = lax.bitcast_convert_type(qep, jnp.int32)
    qcb = jnp.concatenate(
        [qb, qeb, jnp.zeros((N, QW - KW - 8), jnp.int32)], axis=1)
    kb = lax.bitcast_convert_type(
        k.astype(jnp.bfloat16).reshape(N, KW, 2), jnp.int32)
    vp = jnp.transpose(v.astype(jnp.bfloat16).reshape(N, 4, 2, 16),
                       (0, 1, 3, 2))
    vb = lax.bitcast_convert_type(vp, jnp.int32).reshape(N, VW)
    return qcb, kb, vb


def _build_eax(ea):
    z8 = jnp.zeros((E, 8), jnp.float32)
    return jnp.concatenate([ea[:, :8], z8, ea[:, 8:], z8, ea], axis=1)


def kernel(x, edge_index, edge_attr,
           W1, b1, q1W, q1b, k1W, k1b, v1W, v1b, e1W, s1W, s1b,
           W2, b2, q2W, q2b, k2W, k2b, v2W, v2b, e2W, s2W, s2b, W3, b3):
    ei3 = jnp.transpose(edge_index.reshape(2, NCHUNK, CH), (1, 0, 2))
    eax = _build_eax(edge_attr)

    qc1, k1, v1, skip1 = _tc_pre(
        x, W1, b1.reshape(1, C), q1W, q1b.reshape(1, C), k1W,
        k1b.reshape(1, C), v1W, v1b.reshape(1, C), s1W, s1b.reshape(1, C),
        e1W)

    u1 = _sc_edge(*_pack_bf16(qc1, k1, v1), ei3, eax)

    qc2, k2, v2, skip2 = _tc_mid(
        u1, e1W, skip1, W2, b2.reshape(1, C), q2W, q2b.reshape(1, C), k2W,
        k2b.reshape(1, C), v2W, v2b.reshape(1, C), s2W, s2b.reshape(1, C),
        e2W)

    u2 = _sc_edge(*_pack_bf16(qc2, k2, v2), ei3, eax)

    out = _tc_post(u2, e2W, skip2, W3, b3.reshape(1, 1))
    return out.reshape(N)


# speedup vs baseline: 1.2208x; 1.0636x over previous
import functools
import math

import jax
import jax.numpy as jnp
from jax import lax
from jax.experimental import pallas as pl
from jax.experimental.pallas import tpu as pltpu
from jax.experimental.pallas import tpu_sc as plsc

N = 10000
E = 320000
D = 128
ED = 16
C = 128
QC = C + ED
QW = 80
KW = C // 2
VW = C // 2
EAX = 48

NC = 2
NS = 16
NW = NC * NS

CH = 32
NCHUNK = E // CH
ROW = 160
ZCH = 16
NZC = N // ZCH

TB = 1000
GRID = N // TB

_RSQRT_C = 1.0 / math.sqrt(float(C))


def _proj_body(h, qW, qb, kW, kb, vW, vb, sW, sb, eW):
    qs = (jnp.dot(h, qW.T, preferred_element_type=jnp.float32) + qb) * _RSQRT_C
    k = jnp.dot(h, kW.T, preferred_element_type=jnp.float32) + kb
    v = jnp.dot(h, vW.T, preferred_element_type=jnp.float32) + vb
    skip = jnp.dot(h, sW.T, preferred_element_type=jnp.float32) + sb
    qe = jnp.dot(qs, eW, preferred_element_type=jnp.float32)
    return jnp.concatenate([qs, qe], axis=1), k, v, skip


def _tc_pre_body(x_ref, W1_ref, b1_ref, qW_ref, qb_ref, kW_ref, kb_ref,
                 vW_ref, vb_ref, sW_ref, sb_ref, eW_ref,
                 qc_ref, k_ref, v_ref, skip_ref):
    x = x_ref[...]
    h = jnp.maximum(
        jnp.dot(x, W1_ref[...].T, preferred_element_type=jnp.float32)
        + b1_ref[...], 0.0)
    qc, k, v, skip = _proj_body(
        h, qW_ref[...], qb_ref[...], kW_ref[...], kb_ref[...], vW_ref[...],
        vb_ref[...], sW_ref[...], sb_ref[...], eW_ref[...])
    qc_ref[...] = qc
    k_ref[...] = k
    v_ref[...] = v
    skip_ref[...] = skip


def _norm_block(u, eW, skip):
    usum = u[0] + u[1]
    dinv = 1.0 / (usum[:, 144:145] + 1e-16)
    msg = usum[:, 0:128] * dinv
    z = usum[:, 128:144] * dinv
    h1 = msg + jnp.dot(z, eW.T, preferred_element_type=jnp.float32) + skip
    return jnp.maximum(h1, 0.0)


def _tc_mid_body(u_ref, e1W_ref, skip1_ref, W2_ref, b2_ref,
                 qW_ref, qb_ref, kW_ref, kb_ref, vW_ref, vb_ref,
                 sW_ref, sb_ref, e2W_ref,
                 qc_ref, k_ref, v_ref, skip_ref):
    h1 = _norm_block(u_ref[...], e1W_ref[...], skip1_ref[...])
    h = jnp.maximum(
        jnp.dot(h1, W2_ref[...].T, preferred_element_type=jnp.float32)
        + b2_ref[...], 0.0)
    qc, k, v, skip = _proj_body(
        h, qW_ref[...], qb_ref[...], kW_ref[...], kb_ref[...], vW_ref[...],
        vb_ref[...], sW_ref[...], sb_ref[...], e2W_ref[...])
    qc_ref[...] = qc
    k_ref[...] = k
    v_ref[...] = v
    skip_ref[...] = skip


def _tc_post_body(u_ref, e2W_ref, skip2_ref, W3_ref, b3_ref, out_ref):
    h = _norm_block(u_ref[...], e2W_ref[...], skip2_ref[...])
    out_ref[...] = (jnp.sum(h * W3_ref[...], axis=1, keepdims=True)
                    + b3_ref[0, 0])


def _full(shape):
    return pl.BlockSpec(shape, lambda i: tuple(0 for _ in shape))


_W_SPECS = [
    _full((C, C)), _full((1, C)),
    _full((C, C)), _full((1, C)),
    _full((C, C)), _full((1, C)),
    _full((C, C)), _full((1, C)),
    _full((C, ED)),
]

_PROJ_OUT_SPECS = [
    pl.BlockSpec((TB, QC), lambda i: (i, 0)),
    pl.BlockSpec((TB, C), lambda i: (i, 0)),
    pl.BlockSpec((TB, C), lambda i: (i, 0)),
    pl.BlockSpec((TB, C), lambda i: (i, 0)),
]

_PROJ_OUT_SHAPES = [
    jax.ShapeDtypeStruct((N, QC), jnp.float32),
    jax.ShapeDtypeStruct((N, C), jnp.float32),
    jax.ShapeDtypeStruct((N, C), jnp.float32),
    jax.ShapeDtypeStruct((N, C), jnp.float32),
]

_tc_pre = pl.pallas_call(
    _tc_pre_body,
    grid=(GRID,),
    in_specs=[pl.BlockSpec((TB, D), lambda i: (i, 0)),
              _full((C, D)), _full((1, C))] + _W_SPECS,
    out_specs=_PROJ_OUT_SPECS,
    out_shape=_PROJ_OUT_SHAPES,
)

_tc_mid = pl.pallas_call(
    _tc_mid_body,
    grid=(GRID,),
    in_specs=[pl.BlockSpec((NC, TB, ROW), lambda i: (0, i, 0)),
              _full((C, ED)),
              pl.BlockSpec((TB, C), lambda i: (i, 0)),
              _full((C, C)), _full((1, C))] + _W_SPECS,
    out_specs=_PROJ_OUT_SPECS,
    out_shape=_PROJ_OUT_SHAPES,
)

_tc_post = pl.pallas_call(
    _tc_post_body,
    grid=(GRID,),
    in_specs=[pl.BlockSpec((NC, TB, ROW), lambda i: (0, i, 0)),
              _full((C, ED)),
              pl.BlockSpec((TB, C), lambda i: (i, 0)),
              _full((1, C)), _full((1, 1))],
    out_specs=pl.BlockSpec((TB, 1), lambda i: (i, 0)),
    out_shape=jax.ShapeDtypeStruct((N, 1), jnp.float32),
)


_BASE_CHUNKS = NCHUNK // NW
_EXTRA = NCHUNK - _BASE_CHUNKS * NW


def _sc_edge_body(qc_hbm, kv_hbm, ei3_hbm, eax_hbm,
                  out_hbm,
                  idxs, qrows2, kvrows2, eav2, exv, urow2,
                  uacc, semg0, semg1, sems):
    cid = lax.axis_index("c")
    sid = lax.axis_index("s")
    wid = sid * NC + cid

    iot = lax.iota(jnp.int32, 16)
    zeros16 = jnp.zeros((16,), jnp.float32)
    onehot0 = jnp.where(iot == 0, 1.0, 0.0).astype(jnp.float32)
    semg = (semg0, semg1)

    def _zero_row(i, carry):
        for t in range(ROW // 16):
            urow2[0, i, pl.ds(16 * t, 16)] = zeros16
        return carry

    lax.fori_loop(0, ZCH, _zero_row, 0)

    def _zero_chunk(i, carry):
        off = pl.multiple_of((sid + NS * i) * ZCH, 8)
        pltpu.sync_copy(urow2.at[0, pl.ds(0, ZCH)],
                        uacc.at[pl.ds(off, ZCH)])
        return carry

    nzc_mine = (NZC - 1 - sid) // NS + 1
    lax.fori_loop(0, nzc_mine, _zero_chunk, 0)
    plsc.subcore_barrier()

    start = wid * _BASE_CHUNKS + jnp.minimum(wid, _EXTRA)
    nch = _BASE_CHUNKS + jnp.where(wid < _EXTRA, 1, 0)

    def _sidx(ci, which):
        return idxs.at[(ci >> 3) & 1, ci & 7, which]

    maskhi = jnp.full((16,), -65536, jnp.int32)

    def _unpack(w):
        lo = plsc.bitcast(jnp.left_shift(w, 16), jnp.float32)
        hi = plsc.bitcast(jnp.bitwise_and(w, maskhi), jnp.float32)
        return lo, hi

    def _issue_kea(ci, b):
        pltpu.async_copy(kv_hbm.at[_sidx(ci, 0)], kvrows2.at[b], semg[b])
        pltpu.async_copy(eax_hbm.at[pl.ds((start + ci) * CH, CH)],
                         eav2.at[b], semg[b])

    def _process(ci, b):
        @pl.when(ci >= 2)
        def _():
            pltpu.make_async_copy(out_hbm.at[cid, pl.ds(0, CH)],
                                  urow2.at[b], sems).wait()

        pltpu.make_async_copy(qc_hbm.at[pl.ds(0, CH)], qrows2.at[b],
                              semg[b]).wait()
        pltpu.make_async_copy(kv_hbm.at[pl.ds(0, CH)], kvrows2.at[b],
                              semg[b]).wait()
        pltpu.make_async_copy(eax_hbm.at[pl.ds(0, CH)], eav2.at[b],
                              semg[b]).wait()

        def _group_a(g, gcarry):
            alpha = zeros16
            for j in range(16):
                jj = g * 16 + j
                prod = zeros16
                for t in range(KW // 16):
                    qlo, qhi = _unpack(qrows2[b, jj, pl.ds(16 * t, 16)])
                    klo, khi = _unpack(kvrows2[b, jj, pl.ds(16 * t, 16)])
                    prod = prod + qlo * klo + qhi * khi
                qelo, qehi = _unpack(qrows2[b, jj, pl.ds(KW, 16)])
                prod = (prod + qelo * eav2[b, jj, pl.ds(0, 16)]
                        + qehi * eav2[b, jj, pl.ds(16, 16)])
                s = jnp.full((16,), jnp.sum(prod))
                alpha = jnp.where(iot == j, s, alpha)
            exv[pl.ds(g * 16, 16)] = jnp.exp(alpha)
            return gcarry

        lax.fori_loop(0, CH // 16, _group_a, 0)

        @pl.when(jnp.logical_and((ci & 7) == 6, ci + 2 < nch))
        def _():
            nci = ci + 2
            pltpu.sync_copy(ei3_hbm.at[pl.ds(start + nci, 8)],
                            idxs.at[(nci >> 3) & 1])

        @pl.when(ci + 2 < nch)
        def _():
            pltpu.async_copy(qc_hbm.at[_sidx(ci + 2, 1)], qrows2.at[b],
                             semg[b])

        def _group_b(g, gcarry):
            exg = exv[pl.ds(g * 16, 16)]
            for j in range(16):
                jj = g * 16 + j
                exs = jnp.full((16,), exg[j])
                for t in range(VW // 16):
                    vlo, vhi = _unpack(kvrows2[b, jj,
                                               pl.ds(KW + 16 * t, 16)])
                    urow2[b, jj, pl.ds(32 * t, 16)] = exs * vlo
                    urow2[b, jj, pl.ds(32 * t + 16, 16)] = exs * vhi
                urow2[b, jj, pl.ds(C, 16)] = exs * eav2[b, jj, pl.ds(32, 16)]
                urow2[b, jj, pl.ds(C + ED, 16)] = exs * onehot0
            return gcarry

        lax.fori_loop(0, CH // 16, _group_b, 0)

        pltpu.async_copy(urow2.at[b], uacc.at[_sidx(ci, 1)], sems,
                         add=True)

        @pl.when(ci + 2 < nch)
        def _():
            _issue_kea(ci + 2, b)

    pltpu.sync_copy(ei3_hbm.at[pl.ds(start, 8)], idxs.at[0])
    for b0 in (0, 1):
        pltpu.async_copy(qc_hbm.at[_sidx(b0, 1)], qrows2.at[b0], semg[b0])
        _issue_kea(b0, b0)

    def _chunk(ci, carry):
        even = lax.rem(ci, 2) == 0

        @pl.when(even)
        def _():
            _process(ci, 0)

        @pl.when(jnp.logical_not(even))
        def _():
            _process(ci, 1)

        return carry

    lax.fori_loop(0, nch, _chunk, 0)

    pltpu.make_async_copy(out_hbm.at[cid, pl.ds(0, CH)], urow2.at[0],
                          sems).wait()
    pltpu.make_async_copy(out_hbm.at[cid, pl.ds(0, CH)], urow2.at[1],
                          sems).wait()

    plsc.subcore_barrier()

    def _out_chunk(i, carry):
        off = pl.multiple_of((sid + NS * i) * ZCH, 8)
        pltpu.sync_copy(uacc.at[pl.ds(off, ZCH)],
                        out_hbm.at[cid, pl.ds(off, ZCH)])
        return carry

    lax.fori_loop(0, nzc_mine, _out_chunk, 0)


_sc_edge = pl.kernel(
    _sc_edge_body,
    out_type=jax.ShapeDtypeStruct((NC, N, ROW), jnp.float32),
    mesh=plsc.VectorSubcoreMesh(core_axis_name="c", subcore_axis_name="s",
                                num_cores=NC, num_subcores=NS),
    compiler_params=pltpu.CompilerParams(needs_layout_passes=False,
                                         use_tc_tiling_on_sc=False),
    scratch_types=[
        pltpu.VMEM((2, 8, 2, CH), jnp.int32),
        pltpu.VMEM((2, CH, QW), jnp.int32),
        pltpu.VMEM((2, CH, KW + VW), jnp.int32),
        pltpu.VMEM((2, CH, EAX), jnp.float32),
        pltpu.VMEM((CH,), jnp.float32),
        pltpu.VMEM((2, CH, ROW), jnp.float32),
        pltpu.VMEM_SHARED((N, ROW), jnp.float32),
        pltpu.SemaphoreType.DMA,
        pltpu.SemaphoreType.DMA,
        pltpu.SemaphoreType.DMA,
    ],
)


def _pack_bf16(qc, k, v):
    q = qc[:, :C].astype(jnp.bfloat16)
    qe = qc[:, C:].astype(jnp.bfloat16)
    qb = lax.bitcast_convert_type(q.reshape(N, KW, 2), jnp.int32)
    qep = jnp.stack([qe[:, :8], qe[:, 8:]], axis=-1)
    qeb = lax.bitcast_convert_type(qep, jnp.int32)
    qcb = jnp.concatenate(
        [qb, qeb, jnp.zeros((N, QW - KW - 8), jnp.int32)], axis=1)
    kb = lax.bitcast_convert_type(
        k.astype(jnp.bfloat16).reshape(N, KW, 2), jnp.int32)
    vp = jnp.transpose(v.astype(jnp.bfloat16).reshape(N, 4, 2, 16),
                       (0, 1, 3, 2))
    vb = lax.bitcast_convert_type(vp, jnp.int32).reshape(N, VW)
    return qcb, jnp.concatenate([kb, vb], axis=1)


def _build_eax(ea):
    z8 = jnp.zeros((E, 8), jnp.float32)
    return jnp.concatenate([ea[:, :8], z8, ea[:, 8:], z8, ea], axis=1)


def kernel(x, edge_index, edge_attr,
           W1, b1, q1W, q1b, k1W, k1b, v1W, v1b, e1W, s1W, s1b,
           W2, b2, q2W, q2b, k2W, k2b, v2W, v2b, e2W, s2W, s2b, W3, b3):
    ei3 = jnp.transpose(edge_index.reshape(2, NCHUNK, CH), (1, 0, 2))
    eax = _build_eax(edge_attr)

    qc1, k1, v1, skip1 = _tc_pre(
        x, W1, b1.reshape(1, C), q1W, q1b.reshape(1, C), k1W,
        k1b.reshape(1, C), v1W, v1b.reshape(1, C), s1W, s1b.reshape(1, C),
        e1W)

    u1 = _sc_edge(*_pack_bf16(qc1, k1, v1), ei3, eax)

    qc2, k2, v2, skip2 = _tc_mid(
        u1, e1W, skip1, W2, b2.reshape(1, C), q2W, q2b.reshape(1, C), k2W,
        k2b.reshape(1, C), v2W, v2b.reshape(1, C), s2W, s2b.reshape(1, C),
        e2W)

    u2 = _sc_edge(*_pack_bf16(qc2, k2, v2), ei3, eax)

    out = _tc_post(u2, e2W, skip2, W3, b3.reshape(1, 1))
    return out.reshape(N)


# speedup vs baseline: 1.2565x; 1.0292x over previous
import functools
import math

import jax
import jax.numpy as jnp
from jax import lax
from jax.experimental import pallas as pl
from jax.experimental.pallas import tpu as pltpu
from jax.experimental.pallas import tpu_sc as plsc

N = 10000
E = 320000
D = 128
ED = 16
C = 128
QC = C + ED
QW = 80
KW = C // 2
VW = C // 2
EAX = 32

NC = 2
NS = 16
NW = NC * NS

CH = 32
NCHUNK = E // CH
ROW = 160
ZCH = 16
NZC = N // ZCH

TB = 1000
GRID = N // TB

_RSQRT_C = 1.0 / math.sqrt(float(C))


def _proj_body(h, qW, qb, kW, kb, vW, vb, sW, sb, eW):
    qs = (jnp.dot(h, qW.T, preferred_element_type=jnp.float32) + qb) * _RSQRT_C
    k = jnp.dot(h, kW.T, preferred_element_type=jnp.float32) + kb
    v = jnp.dot(h, vW.T, preferred_element_type=jnp.float32) + vb
    skip = jnp.dot(h, sW.T, preferred_element_type=jnp.float32) + sb
    qe = jnp.dot(qs, eW, preferred_element_type=jnp.float32)
    return jnp.concatenate([qs, qe], axis=1), k, v, skip


def _tc_pre_body(x_ref, W1_ref, b1_ref, qW_ref, qb_ref, kW_ref, kb_ref,
                 vW_ref, vb_ref, sW_ref, sb_ref, eW_ref,
                 qc_ref, k_ref, v_ref, skip_ref):
    x = x_ref[...]
    h = jnp.maximum(
        jnp.dot(x, W1_ref[...].T, preferred_element_type=jnp.float32)
        + b1_ref[...], 0.0)
    qc, k, v, skip = _proj_body(
        h, qW_ref[...], qb_ref[...], kW_ref[...], kb_ref[...], vW_ref[...],
        vb_ref[...], sW_ref[...], sb_ref[...], eW_ref[...])
    qc_ref[...] = qc
    k_ref[...] = k
    v_ref[...] = v
    skip_ref[...] = skip


def _norm_block(u, eW, skip):
    usum = u[0] + u[1]
    dinv = 1.0 / (usum[:, 144:145] + 1e-16)
    msg = usum[:, 0:128] * dinv
    z = usum[:, 128:144] * dinv
    h1 = msg + jnp.dot(z, eW.T, preferred_element_type=jnp.float32) + skip
    return jnp.maximum(h1, 0.0)


def _tc_mid_body(u_ref, e1W_ref, skip1_ref, W2_ref, b2_ref,
                 qW_ref, qb_ref, kW_ref, kb_ref, vW_ref, vb_ref,
                 sW_ref, sb_ref, e2W_ref,
                 qc_ref, k_ref, v_ref, skip_ref):
    h1 = _norm_block(u_ref[...], e1W_ref[...], skip1_ref[...])
    h = jnp.maximum(
        jnp.dot(h1, W2_ref[...].T, preferred_element_type=jnp.float32)
        + b2_ref[...], 0.0)
    qc, k, v, skip = _proj_body(
        h, qW_ref[...], qb_ref[...], kW_ref[...], kb_ref[...], vW_ref[...],
        vb_ref[...], sW_ref[...], sb_ref[...], e2W_ref[...])
    qc_ref[...] = qc
    k_ref[...] = k
    v_ref[...] = v
    skip_ref[...] = skip


def _tc_post_body(u_ref, e2W_ref, skip2_ref, W3_ref, b3_ref, out_ref):
    h = _norm_block(u_ref[...], e2W_ref[...], skip2_ref[...])
    out_ref[...] = (jnp.sum(h * W3_ref[...], axis=1, keepdims=True)
                    + b3_ref[0, 0])


def _full(shape):
    return pl.BlockSpec(shape, lambda i: tuple(0 for _ in shape))


_W_SPECS = [
    _full((C, C)), _full((1, C)),
    _full((C, C)), _full((1, C)),
    _full((C, C)), _full((1, C)),
    _full((C, C)), _full((1, C)),
    _full((C, ED)),
]

_PROJ_OUT_SPECS = [
    pl.BlockSpec((TB, QC), lambda i: (i, 0)),
    pl.BlockSpec((TB, C), lambda i: (i, 0)),
    pl.BlockSpec((TB, C), lambda i: (i, 0)),
    pl.BlockSpec((TB, C), lambda i: (i, 0)),
]

_PROJ_OUT_SHAPES = [
    jax.ShapeDtypeStruct((N, QC), jnp.float32),
    jax.ShapeDtypeStruct((N, C), jnp.float32),
    jax.ShapeDtypeStruct((N, C), jnp.float32),
    jax.ShapeDtypeStruct((N, C), jnp.float32),
]

_tc_pre = pl.pallas_call(
    _tc_pre_body,
    grid=(GRID,),
    in_specs=[pl.BlockSpec((TB, D), lambda i: (i, 0)),
              _full((C, D)), _full((1, C))] + _W_SPECS,
    out_specs=_PROJ_OUT_SPECS,
    out_shape=_PROJ_OUT_SHAPES,
)

_tc_mid = pl.pallas_call(
    _tc_mid_body,
    grid=(GRID,),
    in_specs=[pl.BlockSpec((NC, TB, ROW), lambda i: (0, i, 0)),
              _full((C, ED)),
              pl.BlockSpec((TB, C), lambda i: (i, 0)),
              _full((C, C)), _full((1, C))] + _W_SPECS,
    out_specs=_PROJ_OUT_SPECS,
    out_shape=_PROJ_OUT_SHAPES,
)

_tc_post = pl.pallas_call(
    _tc_post_body,
    grid=(GRID,),
    in_specs=[pl.BlockSpec((NC, TB, ROW), lambda i: (0, i, 0)),
              _full((C, ED)),
              pl.BlockSpec((TB, C), lambda i: (i, 0)),
              _full((1, C)), _full((1, 1))],
    out_specs=pl.BlockSpec((TB, 1), lambda i: (i, 0)),
    out_shape=jax.ShapeDtypeStruct((N, 1), jnp.float32),
)


_BASE_CHUNKS = NCHUNK // NW
_EXTRA = NCHUNK - _BASE_CHUNKS * NW


def _sc_edge_body(qc_hbm, kv_hbm, ei3_hbm, eax_hbm,
                  out_hbm,
                  idxs, qrows2, kvrows2, eav2, exv, urow2,
                  uacc, semg0, semg1, sems):
    cid = lax.axis_index("c")
    sid = lax.axis_index("s")
    wid = sid * NC + cid

    iot = lax.iota(jnp.int32, 16)
    zeros16 = jnp.zeros((16,), jnp.float32)
    onehot0 = jnp.where(iot == 0, 1.0, 0.0).astype(jnp.float32)
    semg = (semg0, semg1)

    def _zero_row(i, carry):
        for t in range(ROW // 16):
            urow2[0, i, pl.ds(16 * t, 16)] = zeros16
        return carry

    lax.fori_loop(0, ZCH, _zero_row, 0)

    def _zero_chunk(i, carry):
        off = pl.multiple_of((sid + NS * i) * ZCH, 8)
        pltpu.sync_copy(urow2.at[0, pl.ds(0, ZCH)],
                        uacc.at[pl.ds(off, ZCH)])
        return carry

    nzc_mine = (NZC - 1 - sid) // NS + 1
    lax.fori_loop(0, nzc_mine, _zero_chunk, 0)
    plsc.subcore_barrier()

    start = wid * _BASE_CHUNKS + jnp.minimum(wid, _EXTRA)
    nch = _BASE_CHUNKS + jnp.where(wid < _EXTRA, 1, 0)

    def _sidx(ci, which):
        return idxs.at[(ci >> 3) & 1, ci & 7, which]

    maskhi = jnp.full((16,), -65536, jnp.int32)

    def _unpack(w):
        lo = plsc.bitcast(jnp.left_shift(w, 16), jnp.float32)
        hi = plsc.bitcast(jnp.bitwise_and(w, maskhi), jnp.float32)
        return lo, hi

    def _issue_kea(ci, b):
        pltpu.async_copy(kv_hbm.at[_sidx(ci, 0)], kvrows2.at[b], semg[b])
        pltpu.async_copy(eax_hbm.at[pl.ds((start + ci) * CH, CH)],
                         eav2.at[b], semg[b])

    def _process(ci, b):
        @pl.when(ci >= 2)
        def _():
            pltpu.make_async_copy(out_hbm.at[cid, pl.ds(0, CH)],
                                  urow2.at[b], sems).wait()

        pltpu.make_async_copy(qc_hbm.at[pl.ds(0, CH)], qrows2.at[b],
                              semg[b]).wait()
        pltpu.make_async_copy(kv_hbm.at[pl.ds(0, CH)], kvrows2.at[b],
                              semg[b]).wait()
        pltpu.make_async_copy(eax_hbm.at[pl.ds(0, CH)], eav2.at[b],
                              semg[b]).wait()

        def _group_a(g, gcarry):
            alpha = zeros16
            for j in range(16):
                jj = g * 16 + j
                prod = zeros16
                for t in range(KW // 16):
                    qlo, qhi = _unpack(qrows2[b, jj, pl.ds(16 * t, 16)])
                    klo, khi = _unpack(kvrows2[b, jj, pl.ds(16 * t, 16)])
                    prod = prod + qlo * klo + qhi * khi
                qelo, qehi = _unpack(qrows2[b, jj, pl.ds(KW, 16)])
                prod = (prod + qelo * eav2[b, jj, pl.ds(0, 16)]
                        + qehi * eav2[b, jj, pl.ds(16, 16)])
                s = jnp.full((16,), jnp.sum(prod))
                alpha = jnp.where(iot == j, s, alpha)
            exv[pl.ds(g * 16, 16)] = jnp.exp(alpha)
            return gcarry

        lax.fori_loop(0, CH // 16, _group_a, 0)

        @pl.when(jnp.logical_and((ci & 7) == 6, ci + 2 < nch))
        def _():
            nci = ci + 2
            pltpu.sync_copy(ei3_hbm.at[pl.ds(start + nci, 8)],
                            idxs.at[(nci >> 3) & 1])

        @pl.when(ci + 2 < nch)
        def _():
            pltpu.async_copy(qc_hbm.at[_sidx(ci + 2, 1)], qrows2.at[b],
                             semg[b])

        def _group_b(g, gcarry):
            exg = exv[pl.ds(g * 16, 16)]
            for j in range(16):
                jj = g * 16 + j
                exs = jnp.full((16,), exg[j])
                for t in range(VW // 16):
                    vlo, vhi = _unpack(kvrows2[b, jj,
                                               pl.ds(KW + 16 * t, 16)])
                    urow2[b, jj, pl.ds(32 * t, 16)] = exs * vlo
                    urow2[b, jj, pl.ds(32 * t + 16, 16)] = exs * vhi
                urow2[b, jj, pl.ds(C, 16)] = exs * eav2[b, jj, pl.ds(0, 16)]
                urow2[b, jj, pl.ds(C + ED, 16)] = exs * onehot0
            return gcarry

        lax.fori_loop(0, CH // 16, _group_b, 0)

        pltpu.async_copy(urow2.at[b], uacc.at[_sidx(ci, 1)], sems,
                         add=True)

        @pl.when(ci + 2 < nch)
        def _():
            _issue_kea(ci + 2, b)

    pltpu.sync_copy(ei3_hbm.at[pl.ds(start, 8)], idxs.at[0])
    for b0 in (0, 1):
        pltpu.async_copy(qc_hbm.at[_sidx(b0, 1)], qrows2.at[b0], semg[b0])
        _issue_kea(b0, b0)

    def _chunk(ci, carry):
        even = lax.rem(ci, 2) == 0

        @pl.when(even)
        def _():
            _process(ci, 0)

        @pl.when(jnp.logical_not(even))
        def _():
            _process(ci, 1)

        return carry

    lax.fori_loop(0, nch, _chunk, 0)

    pltpu.make_async_copy(out_hbm.at[cid, pl.ds(0, CH)], urow2.at[0],
                          sems).wait()
    pltpu.make_async_copy(out_hbm.at[cid, pl.ds(0, CH)], urow2.at[1],
                          sems).wait()

    plsc.subcore_barrier()

    def _out_chunk(i, carry):
        off = pl.multiple_of((sid + NS * i) * ZCH, 8)
        pltpu.sync_copy(uacc.at[pl.ds(off, ZCH)],
                        out_hbm.at[cid, pl.ds(off, ZCH)])
        return carry

    lax.fori_loop(0, nzc_mine, _out_chunk, 0)


_sc_edge = pl.kernel(
    _sc_edge_body,
    out_type=jax.ShapeDtypeStruct((NC, N, ROW), jnp.float32),
    mesh=plsc.VectorSubcoreMesh(core_axis_name="c", subcore_axis_name="s",
                                num_cores=NC, num_subcores=NS),
    compiler_params=pltpu.CompilerParams(needs_layout_passes=False,
                                         use_tc_tiling_on_sc=False),
    scratch_types=[
        pltpu.VMEM((2, 8, 2, CH), jnp.int32),
        pltpu.VMEM((2, CH, QW), jnp.int32),
        pltpu.VMEM((2, CH, KW + VW), jnp.int32),
        pltpu.VMEM((2, CH, EAX), jnp.float32),
        pltpu.VMEM((CH,), jnp.float32),
        pltpu.VMEM((2, CH, ROW), jnp.float32),
        pltpu.VMEM_SHARED((N, ROW), jnp.float32),
        pltpu.SemaphoreType.DMA,
        pltpu.SemaphoreType.DMA,
        pltpu.SemaphoreType.DMA,
    ],
)


def _pack_bf16(qc, k, v):
    q = qc[:, :C].astype(jnp.bfloat16)
    qe = qc[:, C:].astype(jnp.bfloat16)
    qb = lax.bitcast_convert_type(q.reshape(N, KW, 2), jnp.int32)
    qep = jnp.stack([qe[:, :8], qe[:, 8:]], axis=-1)
    qeb = lax.bitcast_convert_type(qep, jnp.int32)
    qcb = jnp.concatenate(
        [qb, qeb, jnp.zeros((N, QW - KW - 8), jnp.int32)], axis=1)
    kb = lax.bitcast_convert_type(
        k.astype(jnp.bfloat16).reshape(N, KW, 2), jnp.int32)
    vp = jnp.transpose(v.astype(jnp.bfloat16).reshape(N, 4, 2, 16),
                       (0, 1, 3, 2))
    vb = lax.bitcast_convert_type(vp, jnp.int32).reshape(N, VW)
    return qcb, jnp.concatenate([kb, vb], axis=1)


def _build_eax(ea):
    z8 = jnp.zeros((E, 8), jnp.float32)
    return jnp.concatenate([ea, ea[:, 8:], z8], axis=1)


def kernel(x, edge_index, edge_attr,
           W1, b1, q1W, q1b, k1W, k1b, v1W, v1b, e1W, s1W, s1b,
           W2, b2, q2W, q2b, k2W, k2b, v2W, v2b, e2W, s2W, s2b, W3, b3):
    ei3 = jnp.transpose(edge_index.reshape(2, NCHUNK, CH), (1, 0, 2))
    eax = _build_eax(edge_attr)

    qc1, k1, v1, skip1 = _tc_pre(
        x, W1, b1.reshape(1, C), q1W, q1b.reshape(1, C), k1W,
        k1b.reshape(1, C), v1W, v1b.reshape(1, C), s1W, s1b.reshape(1, C),
        e1W)

    u1 = _sc_edge(*_pack_bf16(qc1, k1, v1), ei3, eax)

    qc2, k2, v2, skip2 = _tc_mid(
        u1, e1W, skip1, W2, b2.reshape(1, C), q2W, q2b.reshape(1, C), k2W,
        k2b.reshape(1, C), v2W, v2b.reshape(1, C), s2W, s2b.reshape(1, C),
        e2W)

    u2 = _sc_edge(*_pack_bf16(qc2, k2, v2), ei3, eax)

    out = _tc_post(u2, e2W, skip2, W3, b3.reshape(1, 1))
    return out.reshape(N)
